# baseline (speedup 1.0000x reference)
import jax
import jax.numpy as jnp
from jax.experimental import pallas as pl
from jax.experimental.pallas import tpu as pltpu

_N = 10000
_R_BLOCK = 400
_G_BLOCK = 256
_N_IDX = 2500
_N_IDX_PAD = 2560


def _fused_kernel(idx_ref, w0_ref, w1_ref, adj_hbm, x_hbm, o_ref,
                  buf, xw0, hw1, x_s, sems):
    i = pl.program_id(0)
    n_rb = _N // _R_BLOCK
    n_gb = _N_IDX_PAD // _G_BLOCK

    def issue_l1(block, slot):
        pltpu.make_async_copy(
            adj_hbm.at[pl.ds(block * _R_BLOCK, _R_BLOCK), :],
            buf.at[slot, pl.ds(0, _R_BLOCK), :],
            sems.at[slot],
        ).start()

    def issue_l1_half(half):
        pltpu.make_async_copy(
            adj_hbm.at[pl.ds(half * (_R_BLOCK // 2), _R_BLOCK // 2), :],
            buf.at[0, pl.ds(half * (_R_BLOCK // 2), _R_BLOCK // 2), :],
            sems.at[0],
        ).start()

    def issue_gather(block, slot, count):
        def body(r, carry):
            row = idx_ref[block * _G_BLOCK + r]
            pltpu.make_async_copy(
                adj_hbm.at[pl.ds(row, 1), :],
                buf.at[slot, pl.ds(r, 1), :],
                sems.at[slot],
            ).start()
            return carry
        jax.lax.fori_loop(0, count, body, 0, unroll=64)

    def wait_rows(slot, nrows):
        pltpu.make_async_copy(
            adj_hbm.at[pl.ds(0, nrows), :],
            buf.at[slot, pl.ds(0, nrows), :],
            sems.at[slot],
        ).wait()

    n_last_rows = 200
    i_last = n_rb + n_gb - 1

    @pl.when(i == 0)
    def _prologue():
        issue_l1_half(0)
        issue_l1_half(1)
        x_copy = pltpu.make_async_copy(x_hbm, x_s, sems.at[2])
        x_copy.start()
        x_copy.wait()
        xw0[...] = jnp.dot(x_s[...], w0_ref[...],
                           preferred_element_type=jnp.float32
                           ).astype(jnp.bfloat16)

    nxt = i + 1
    slot_nxt = jax.lax.rem(nxt, 2)

    @pl.when(nxt < n_rb)
    def _issue_next_l1():
        issue_l1(nxt, slot_nxt)

    @pl.when((nxt >= n_rb) & (nxt < i_last))
    def _issue_next_gather():
        issue_gather(nxt - n_rb, slot_nxt, _G_BLOCK)

    @pl.when(nxt == i_last)
    def _issue_last_gather():
        issue_gather(n_gb - 1, slot_nxt, n_last_rows)

    slot = jax.lax.rem(i, 2)
    half = _R_BLOCK // 2

    def _l1_mm(rbase, nrows):
        h = jnp.maximum(
            jnp.dot(buf[slot, pl.ds(rbase, nrows), :].astype(jnp.bfloat16),
                    xw0[...], preferred_element_type=jnp.float32), 0.0)
        hw1[pl.ds(i * _R_BLOCK + rbase, nrows), :] = jnp.dot(
            h.astype(jnp.bfloat16), w1_ref[...].astype(jnp.bfloat16),
            preferred_element_type=jnp.float32)

    @pl.when(i == 0)
    def _layer1_first():
        wait_rows(slot, half)
        _l1_mm(0, half)
        wait_rows(slot, half)
        _l1_mm(half, half)

    @pl.when((i > 0) & (i < n_rb))
    def _layer1_step():
        wait_rows(slot, _R_BLOCK)
        _l1_mm(0, _R_BLOCK)

    @pl.when((i >= n_rb) & (i < i_last))
    def _gather_mm_step():
        wait_rows(slot, _G_BLOCK)
        o_ref[...] = jnp.dot(buf[slot, pl.ds(0, _G_BLOCK), :], hw1[...],
                             preferred_element_type=jnp.float32)

    @pl.when(i == i_last)
    def _gather_mm_last():
        wait_rows(slot, n_last_rows)
        o_ref[...] = jnp.dot(buf[slot, pl.ds(0, _G_BLOCK), :], hw1[...],
                             preferred_element_type=jnp.float32)


def kernel(x, adj, idx, W0, W1):
    n_rb = _N // _R_BLOCK
    n_gb = _N_IDX_PAD // _G_BLOCK

    idx32 = idx.astype(jnp.int32)
    idx_pad = jnp.concatenate(
        [idx32, jnp.zeros((_N_IDX_PAD - idx32.shape[0],), jnp.int32)])

    out_pad = pl.pallas_call(
        _fused_kernel,
        grid_spec=pltpu.PrefetchScalarGridSpec(
            num_scalar_prefetch=1,
            grid=(n_rb + n_gb,),
            in_specs=[
                pl.BlockSpec(W0.shape, lambda i, idx_ref: (0, 0)),
                pl.BlockSpec(W1.shape, lambda i, idx_ref: (0, 0)),
                pl.BlockSpec(memory_space=pl.ANY),
                pl.BlockSpec(memory_space=pl.ANY),
            ],
            out_specs=pl.BlockSpec(
                (_G_BLOCK, W1.shape[1]),
                lambda i, idx_ref: (jnp.maximum(i - (_N // _R_BLOCK), 0), 0)),
            scratch_shapes=[
                pltpu.VMEM((2, max(_R_BLOCK, _G_BLOCK), _N), jnp.float32),
                pltpu.VMEM((_N, W0.shape[1]), jnp.bfloat16),
                pltpu.VMEM((_N, W1.shape[1]), jnp.float32),
                pltpu.VMEM((_N, W0.shape[0]), jnp.float32),
                pltpu.SemaphoreType.DMA((3,)),
            ],
        ),
        out_shape=jax.ShapeDtypeStruct((_N_IDX_PAD, W1.shape[1]),
                                       jnp.float32),
    )(idx_pad, W0, W1, adj, x)

    return out_pad[:idx.shape[0]]


# speedup vs baseline: 1.0115x; 1.0115x over previous
import jax
import jax.numpy as jnp
from jax.experimental import pallas as pl
from jax.experimental.pallas import tpu as pltpu

_N = 10000
_R_BLOCK = 400
_G_BLOCK = 256
_N_IDX = 2500
_N_IDX_PAD = 2560


def _fused_kernel(idx_ref, x_ref, w0_ref, w1_ref, adj_hbm, o_ref,
                  buf, xw0, hw1, sems):
    i = pl.program_id(0)
    n_rb = _N // _R_BLOCK
    n_gb = _N_IDX_PAD // _G_BLOCK

    def issue_l1(block, slot):
        pltpu.make_async_copy(
            adj_hbm.at[pl.ds(block * _R_BLOCK, _R_BLOCK), :],
            buf.at[slot, pl.ds(0, _R_BLOCK), :],
            sems.at[slot],
        ).start()

    def issue_l1_half(half):
        pltpu.make_async_copy(
            adj_hbm.at[pl.ds(half * (_R_BLOCK // 2), _R_BLOCK // 2), :],
            buf.at[0, pl.ds(half * (_R_BLOCK // 2), _R_BLOCK // 2), :],
            sems.at[0],
        ).start()

    def issue_gather(block, slot, count):
        def body(r, carry):
            row = idx_ref[block * _G_BLOCK + r]
            pltpu.make_async_copy(
                adj_hbm.at[pl.ds(row, 1), :],
                buf.at[slot, pl.ds(r, 1), :],
                sems.at[slot],
            ).start()
            return carry
        jax.lax.fori_loop(0, count, body, 0, unroll=64)

    def wait_rows(slot, nrows):
        pltpu.make_async_copy(
            adj_hbm.at[pl.ds(0, nrows), :],
            buf.at[slot, pl.ds(0, nrows), :],
            sems.at[slot],
        ).wait()

    n_last_rows = 200
    i_last = n_rb + n_gb - 1

    @pl.when(i == 0)
    def _prologue():
        issue_l1_half(0)
        issue_l1_half(1)
        xw0[...] = jnp.dot(x_ref[...], w0_ref[...],
                           preferred_element_type=jnp.float32
                           ).astype(jnp.bfloat16)

    nxt = i + 1
    slot_nxt = jax.lax.rem(nxt, 2)

    @pl.when(nxt < n_rb)
    def _issue_next_l1():
        issue_l1(nxt, slot_nxt)

    @pl.when((nxt >= n_rb) & (nxt < i_last))
    def _issue_next_gather():
        issue_gather(nxt - n_rb, slot_nxt, _G_BLOCK)

    @pl.when(nxt == i_last)
    def _issue_last_gather():
        issue_gather(n_gb - 1, slot_nxt, n_last_rows)

    slot = jax.lax.rem(i, 2)
    half = _R_BLOCK // 2

    def _l1_mm(rbase, nrows):
        h = jnp.maximum(
            jnp.dot(buf[slot, pl.ds(rbase, nrows), :].astype(jnp.bfloat16),
                    xw0[...], preferred_element_type=jnp.float32), 0.0)
        hw1[pl.ds(i * _R_BLOCK + rbase, nrows), :] = jnp.dot(
            h.astype(jnp.bfloat16), w1_ref[...].astype(jnp.bfloat16),
            preferred_element_type=jnp.float32)

    @pl.when(i == 0)
    def _layer1_first():
        wait_rows(slot, half)
        _l1_mm(0, half)
        wait_rows(slot, half)
        _l1_mm(half, half)

    @pl.when((i > 0) & (i < n_rb))
    def _layer1_step():
        wait_rows(slot, _R_BLOCK)
        _l1_mm(0, _R_BLOCK)

    @pl.when((i >= n_rb) & (i < i_last))
    def _gather_mm_step():
        wait_rows(slot, _G_BLOCK)
        o_ref[...] = jnp.dot(buf[slot, pl.ds(0, _G_BLOCK), :], hw1[...],
                             preferred_element_type=jnp.float32)

    @pl.when(i == i_last)
    def _gather_mm_last():
        wait_rows(slot, n_last_rows)
        o_ref[...] = jnp.dot(buf[slot, pl.ds(0, _G_BLOCK), :], hw1[...],
                             preferred_element_type=jnp.float32)


def kernel(x, adj, idx, W0, W1):
    n_rb = _N // _R_BLOCK
    n_gb = _N_IDX_PAD // _G_BLOCK

    idx32 = idx.astype(jnp.int32)
    idx_pad = jnp.concatenate(
        [idx32, jnp.zeros((_N_IDX_PAD - idx32.shape[0],), jnp.int32)])

    out_pad = pl.pallas_call(
        _fused_kernel,
        grid_spec=pltpu.PrefetchScalarGridSpec(
            num_scalar_prefetch=1,
            grid=(n_rb + n_gb,),
            in_specs=[
                pl.BlockSpec((_N, W0.shape[0]), lambda i, idx_ref: (0, 0)),
                pl.BlockSpec(W0.shape, lambda i, idx_ref: (0, 0)),
                pl.BlockSpec(W1.shape, lambda i, idx_ref: (0, 0)),
                pl.BlockSpec(memory_space=pl.ANY),
            ],
            out_specs=pl.BlockSpec(
                (_G_BLOCK, W1.shape[1]),
                lambda i, idx_ref: (jnp.maximum(i - (_N // _R_BLOCK), 0), 0)),
            scratch_shapes=[
                pltpu.VMEM((2, max(_R_BLOCK, _G_BLOCK), _N), jnp.float32),
                pltpu.VMEM((_N, W0.shape[1]), jnp.bfloat16),
                pltpu.VMEM((_N, W1.shape[1]), jnp.float32),
                pltpu.SemaphoreType.DMA((2,)),
            ],
        ),
        out_shape=jax.ShapeDtypeStruct((_N_IDX_PAD, W1.shape[1]),
                                       jnp.float32),
    )(idx_pad, x, W0, W1, adj)

    return out_pad[:idx.shape[0]]


# speedup vs baseline: 1.0118x; 1.0003x over previous
import jax
import jax.numpy as jnp
from jax.experimental import pallas as pl
from jax.experimental.pallas import tpu as pltpu

_N = 10000
_R_BLOCK = 400
_G_BLOCK = 512
_N_IDX = 2500
_N_IDX_PAD = 2560


def _fused_kernel(idx_ref, x_ref, w0_ref, w1_ref, adj_hbm, o_ref,
                  buf, xw0, hw1, sems):
    i = pl.program_id(0)
    n_rb = _N // _R_BLOCK
    n_gb = _N_IDX_PAD // _G_BLOCK

    def issue_l1(block, slot):
        pltpu.make_async_copy(
            adj_hbm.at[pl.ds(block * _R_BLOCK, _R_BLOCK), :],
            buf.at[slot, pl.ds(0, _R_BLOCK), :],
            sems.at[slot],
        ).start()

    def issue_l1_half(half):
        pltpu.make_async_copy(
            adj_hbm.at[pl.ds(half * (_R_BLOCK // 2), _R_BLOCK // 2), :],
            buf.at[0, pl.ds(half * (_R_BLOCK // 2), _R_BLOCK // 2), :],
            sems.at[0],
        ).start()

    def issue_gather(block, slot, count):
        def body(r, carry):
            row = idx_ref[block * _G_BLOCK + r]
            pltpu.make_async_copy(
                adj_hbm.at[pl.ds(row, 1), :],
                buf.at[slot, pl.ds(r, 1), :],
                sems.at[slot],
            ).start()
            return carry
        jax.lax.fori_loop(0, count, body, 0, unroll=64)

    def wait_rows(slot, nrows):
        pltpu.make_async_copy(
            adj_hbm.at[pl.ds(0, nrows), :],
            buf.at[slot, pl.ds(0, nrows), :],
            sems.at[slot],
        ).wait()

    n_last_rows = 456
    i_last = n_rb + n_gb - 1

    @pl.when(i == 0)
    def _prologue():
        issue_l1_half(0)
        issue_l1_half(1)
        xw0[...] = jnp.dot(x_ref[...], w0_ref[...],
                           preferred_element_type=jnp.float32
                           ).astype(jnp.bfloat16)

    nxt = i + 1
    slot_nxt = jax.lax.rem(nxt, 2)

    @pl.when(nxt < n_rb)
    def _issue_next_l1():
        issue_l1(nxt, slot_nxt)

    @pl.when((nxt >= n_rb) & (nxt < i_last))
    def _issue_next_gather():
        issue_gather(nxt - n_rb, slot_nxt, _G_BLOCK)

    @pl.when(nxt == i_last)
    def _issue_last_gather():
        issue_gather(n_gb - 1, slot_nxt, n_last_rows)

    slot = jax.lax.rem(i, 2)
    half = _R_BLOCK // 2

    def _l1_mm(rbase, nrows):
        h = jnp.maximum(
            jnp.dot(buf[slot, pl.ds(rbase, nrows), :].astype(jnp.bfloat16),
                    xw0[...], preferred_element_type=jnp.float32), 0.0)
        hw1[pl.ds(i * _R_BLOCK + rbase, nrows), :] = jnp.dot(
            h.astype(jnp.bfloat16), w1_ref[...].astype(jnp.bfloat16),
            preferred_element_type=jnp.float32)

    @pl.when(i == 0)
    def _layer1_first():
        wait_rows(slot, half)
        _l1_mm(0, half)
        wait_rows(slot, half)
        _l1_mm(half, half)

    @pl.when((i > 0) & (i < n_rb))
    def _layer1_step():
        wait_rows(slot, _R_BLOCK)
        _l1_mm(0, _R_BLOCK)

    @pl.when((i >= n_rb) & (i < i_last))
    def _gather_mm_step():
        wait_rows(slot, _G_BLOCK)
        o_ref[...] = jnp.dot(buf[slot, pl.ds(0, _G_BLOCK), :], hw1[...],
                             preferred_element_type=jnp.float32)

    @pl.when(i == i_last)
    def _gather_mm_last():
        wait_rows(slot, n_last_rows)
        o_ref[...] = jnp.dot(buf[slot, pl.ds(0, _G_BLOCK), :], hw1[...],
                             preferred_element_type=jnp.float32)


def kernel(x, adj, idx, W0, W1):
    n_rb = _N // _R_BLOCK
    n_gb = _N_IDX_PAD // _G_BLOCK

    idx32 = idx.astype(jnp.int32)
    idx_pad = jnp.concatenate(
        [idx32, jnp.zeros((_N_IDX_PAD - idx32.shape[0],), jnp.int32)])

    out_pad = pl.pallas_call(
        _fused_kernel,
        grid_spec=pltpu.PrefetchScalarGridSpec(
            num_scalar_prefetch=1,
            grid=(n_rb + n_gb,),
            in_specs=[
                pl.BlockSpec((_N, W0.shape[0]), lambda i, idx_ref: (0, 0)),
                pl.BlockSpec(W0.shape, lambda i, idx_ref: (0, 0)),
                pl.BlockSpec(W1.shape, lambda i, idx_ref: (0, 0)),
                pl.BlockSpec(memory_space=pl.ANY),
            ],
            out_specs=pl.BlockSpec(
                (_G_BLOCK, W1.shape[1]),
                lambda i, idx_ref: (jnp.maximum(i - (_N // _R_BLOCK), 0), 0)),
            scratch_shapes=[
                pltpu.VMEM((2, max(_R_BLOCK, _G_BLOCK), _N), jnp.float32),
                pltpu.VMEM((_N, W0.shape[1]), jnp.bfloat16),
                pltpu.VMEM((_N, W1.shape[1]), jnp.float32),
                pltpu.SemaphoreType.DMA((2,)),
            ],
        ),
        out_shape=jax.ShapeDtypeStruct((_N_IDX_PAD, W1.shape[1]),
                                       jnp.float32),
    )(idx_pad, x, W0, W1, adj)

    return out_pad[:idx.shape[0]]
